# Initial kernel scaffold; baseline (speedup 1.0000x reference)
import functools

import jax
import jax.numpy as jnp
from jax import lax
from jax.experimental import pallas as pl
from jax.experimental.pallas import tpu as pltpu
from jax.experimental.pallas import tpu_sc as plsc

N = 10000
E = 320000
D = 128

NC = 2
NS = 16
NW = NC * NS

EPT = E // NW
CH = 128
NFULL = EPT // CH
TAIL = EPT - NFULL * CH

RPT = N // NS
ZR = 125

_sc_mesh = plsc.VectorSubcoreMesh(
    core_axis_name="c", subcore_axis_name="s", num_cores=NC, num_subcores=NS)


def _fill_const(ref, nrows, ncols, value):
    vals = jnp.full((16,), value, jnp.float32)

    def body(i, _):
        for j in range(ncols // 16):
            ref[i, pl.ds(16 * j, 16)] = vals
        return 0

    lax.fori_loop(0, nrows, body, 0)


@functools.partial(
    pl.kernel,
    out_type=jax.ShapeDtypeStruct((NC, N, 16), jnp.float32),
    mesh=_sc_mesh,
    scratch_types=dict(
        cnt_sh=pltpu.VMEM_SHARED((N, 16), jnp.float32),
        ones_v=pltpu.VMEM((CH, 16), jnp.float32),
        idx_v=pltpu.VMEM((CH,), jnp.int32),
        idx_t=pltpu.VMEM((TAIL,), jnp.int32),
        zrow_v=pltpu.VMEM((RPT, 16), jnp.float32),
    ),
)
def _sc_count(dst_hbm, out_hbm, cnt_sh, ones_v, idx_v, idx_t, zrow_v):
    c = lax.axis_index("c")
    s = lax.axis_index("s")
    wid = c * NS + s
    _fill_const(ones_v, CH, 16, 1.0)
    _fill_const(zrow_v, RPT, 16, 0.0)
    row0 = s * RPT
    pltpu.sync_copy(zrow_v, cnt_sh.at[pl.ds(row0, RPT)])
    plsc.subcore_barrier()
    base0 = wid * EPT

    def chunk(i, _):
        pltpu.sync_copy(dst_hbm.at[pl.ds(base0 + i * CH, CH)], idx_v)
        pltpu.sync_copy(ones_v, cnt_sh.at[idx_v], add=True)
        return 0

    lax.fori_loop(0, NFULL, chunk, 0)
    pltpu.sync_copy(dst_hbm.at[pl.ds(base0 + NFULL * CH, TAIL)], idx_t)
    pltpu.sync_copy(ones_v.at[pl.ds(0, TAIL)], cnt_sh.at[idx_t], add=True)
    plsc.subcore_barrier()
    pltpu.sync_copy(cnt_sh.at[pl.ds(row0, RPT)], out_hbm.at[c, pl.ds(row0, RPT)])


@functools.partial(
    pl.kernel,
    out_type=jax.ShapeDtypeStruct((NC, N, D), jnp.float32),
    mesh=_sc_mesh,
    scratch_types=dict(
        acc_sh=pltpu.VMEM_SHARED((N, D), jnp.float32),
        rows_v=pltpu.VMEM((CH, D), jnp.float32),
        rows_t=pltpu.VMEM((TAIL, D), jnp.float32),
        sidx_v=pltpu.VMEM((CH,), jnp.int32),
        didx_v=pltpu.VMEM((CH,), jnp.int32),
        sidx_t=pltpu.VMEM((TAIL,), jnp.int32),
        didx_t=pltpu.VMEM((TAIL,), jnp.int32),
        zrow_v=pltpu.VMEM((ZR, D), jnp.float32),
        sem=pltpu.SemaphoreType.DMA,
    ),
)
def _sc_scatter(g_hbm, src_hbm, dst_hbm, out_hbm, acc_sh, rows_v, rows_t,
                sidx_v, didx_v, sidx_t, didx_t, zrow_v, sem):
    c = lax.axis_index("c")
    s = lax.axis_index("s")
    wid = c * NS + s
    _fill_const(zrow_v, ZR, D, 0.0)
    row0 = s * RPT
    for j in range(RPT // ZR):
        pltpu.sync_copy(zrow_v, acc_sh.at[pl.ds(row0 + j * ZR, ZR)])
    plsc.subcore_barrier()
    base0 = wid * EPT

    def chunk(i, _):
        b = base0 + i * CH
        pltpu.sync_copy(src_hbm.at[pl.ds(b, CH)], sidx_v)
        pltpu.sync_copy(dst_hbm.at[pl.ds(b, CH)], didx_v)
        pltpu.async_copy(g_hbm.at[sidx_v], rows_v, sem).wait()
        pltpu.sync_copy(rows_v, acc_sh.at[didx_v], add=True)
        return 0

    lax.fori_loop(0, NFULL, chunk, 0)
    bt = base0 + NFULL * CH
    pltpu.sync_copy(src_hbm.at[pl.ds(bt, TAIL)], sidx_t)
    pltpu.sync_copy(dst_hbm.at[pl.ds(bt, TAIL)], didx_t)
    pltpu.async_copy(g_hbm.at[sidx_t], rows_t, sem).wait()
    pltpu.sync_copy(rows_t, acc_sh.at[didx_t], add=True)
    plsc.subcore_barrier()
    pltpu.sync_copy(acc_sh.at[pl.ds(row0, RPT)], out_hbm.at[c, pl.ds(row0, RPT)])


_BM = 2000
_GRID = N // _BM


def _dinv_body(degp_ref, out_ref):
    deg = degp_ref[0] + degp_ref[1] + 1.0
    out_ref[...] = lax.rsqrt(deg)


def _tc_dinv(degp):
    return pl.pallas_call(
        _dinv_body,
        out_shape=jax.ShapeDtypeStruct((N, 16), jnp.float32),
    )(degp)


def _mm1_body(x_ref, w_ref, dinv_ref, out_ref):
    di = dinv_ref[:, 0:1]
    out_ref[...] = jnp.dot(
        x_ref[...], w_ref[...], preferred_element_type=jnp.float32) * di


def _tc_mm1(x, W1, dinv16):
    return pl.pallas_call(
        _mm1_body,
        grid=(_GRID,),
        in_specs=[
            pl.BlockSpec((_BM, D), lambda i: (i, 0)),
            pl.BlockSpec((D, D), lambda i: (0, 0)),
            pl.BlockSpec((_BM, 16), lambda i: (i, 0)),
        ],
        out_specs=pl.BlockSpec((_BM, D), lambda i: (i, 0)),
        out_shape=jax.ShapeDtypeStruct((N, D), jnp.float32),
    )(x, W1, dinv16)


def _mm2_body(s_ref, g_ref, dinv_ref, b_ref, w_ref, out_ref):
    di = dinv_ref[:, 0:1]
    pre = (s_ref[0] + s_ref[1] + g_ref[...]) * di + b_ref[...]
    h = jnp.maximum(pre, 0.0)
    out_ref[...] = jnp.dot(
        h, w_ref[...], preferred_element_type=jnp.float32) * di


def _tc_mm2(S1, g1, dinv16, b, W):
    return pl.pallas_call(
        _mm2_body,
        grid=(_GRID,),
        in_specs=[
            pl.BlockSpec((NC, _BM, D), lambda i: (0, i, 0)),
            pl.BlockSpec((_BM, D), lambda i: (i, 0)),
            pl.BlockSpec((_BM, 16), lambda i: (i, 0)),
            pl.BlockSpec((1, D), lambda i: (0, 0)),
            pl.BlockSpec((D, D), lambda i: (0, 0)),
        ],
        out_specs=pl.BlockSpec((_BM, D), lambda i: (i, 0)),
        out_shape=jax.ShapeDtypeStruct((N, D), jnp.float32),
    )(S1, g1, dinv16, b, W)


def _mm3_body(s_ref, g_ref, dinv_ref, b_ref, wfc_ref, bfc_ref, out_ref):
    di = dinv_ref[:, 0:1]
    pre = (s_ref[0] + s_ref[1] + g_ref[...]) * di + b_ref[...]
    h = jnp.maximum(pre, 0.0)
    z = jnp.dot(h, wfc_ref[...], preferred_element_type=jnp.float32)
    out_ref[...] = jax.nn.sigmoid(z + bfc_ref[...])


def _tc_mm3(S2, g2, dinv16, b, Wfc, bfc):
    return pl.pallas_call(
        _mm3_body,
        grid=(_GRID,),
        in_specs=[
            pl.BlockSpec((NC, _BM, D), lambda i: (0, i, 0)),
            pl.BlockSpec((_BM, D), lambda i: (i, 0)),
            pl.BlockSpec((_BM, 16), lambda i: (i, 0)),
            pl.BlockSpec((1, D), lambda i: (0, 0)),
            pl.BlockSpec((D, 1), lambda i: (0, 0)),
            pl.BlockSpec((1, 1), lambda i: (0, 0)),
        ],
        out_specs=pl.BlockSpec((_BM, 1), lambda i: (i, 0)),
        out_shape=jax.ShapeDtypeStruct((N, 1), jnp.float32),
    )(S2, g2, dinv16, b, Wfc, bfc)


def kernel(x, edge_index, W1, b1, W2, b2, Wfc, bfc):
    ei = edge_index.astype(jnp.int32)
    src = ei[0]
    dst = ei[1]
    degp = _sc_count(dst)
    dinv16 = _tc_dinv(degp)
    g1 = _tc_mm1(x, W1, dinv16)
    S1 = _sc_scatter(g1, src, dst)
    g2 = _tc_mm2(S1, g1, dinv16, b1.reshape(1, D), W2)
    S2 = _sc_scatter(g2, src, dst)
    out = _tc_mm3(S2, g2, dinv16, b2.reshape(1, D), Wfc, bfc.reshape(1, 1))
    return out


# speedup vs baseline: 16.1326x; 16.1326x over previous
import functools

import jax
import jax.numpy as jnp
from jax import lax
from jax.experimental import pallas as pl
from jax.experimental.pallas import tpu as pltpu
from jax.experimental.pallas import tpu_sc as plsc

N = 10000
NP = 10240
E = 320000
D = 128

NC = 2
NS = 16
NW = NC * NS

EPT = E // NW
CH = 128
NFULL = EPT // CH
TAIL = EPT - NFULL * CH

RPT = NP // NS
ZR = 128

_sc_mesh = plsc.VectorSubcoreMesh(
    core_axis_name="c", subcore_axis_name="s", num_cores=NC, num_subcores=NS)


def _fill_const(ref, nrows, ncols, value):
    vals = jnp.full((16,), value, jnp.float32)

    def body(i, _):
        for j in range(ncols // 16):
            ref[i, pl.ds(16 * j, 16)] = vals
        return 0

    lax.fori_loop(0, nrows, body, 0)


@functools.partial(
    pl.kernel,
    out_type=jax.ShapeDtypeStruct((NC, NP, 16), jnp.float32),
    mesh=_sc_mesh,
    scratch_types=dict(
        cnt_sh=pltpu.VMEM_SHARED((NP, 16), jnp.float32),
        ones_v=pltpu.VMEM((CH, 16), jnp.float32),
        idx_v=pltpu.VMEM((CH,), jnp.int32),
        idx_t=pltpu.VMEM((TAIL,), jnp.int32),
        zrow_v=pltpu.VMEM((RPT, 16), jnp.float32),
    ),
)
def _sc_count(dst_hbm, out_hbm, cnt_sh, ones_v, idx_v, idx_t, zrow_v):
    c = lax.axis_index("c")
    s = lax.axis_index("s")
    wid = c * NS + s
    _fill_const(ones_v, CH, 16, 1.0)
    _fill_const(zrow_v, RPT, 16, 0.0)
    row0 = s * RPT
    pltpu.sync_copy(zrow_v, cnt_sh.at[pl.ds(row0, RPT)])
    plsc.subcore_barrier()
    base0 = wid * EPT

    def chunk(i, _):
        pltpu.sync_copy(dst_hbm.at[pl.ds(base0 + i * CH, CH)], idx_v)
        pltpu.sync_copy(ones_v, cnt_sh.at[idx_v], add=True)
        return 0

    lax.fori_loop(0, NFULL, chunk, 0)
    pltpu.sync_copy(dst_hbm.at[pl.ds(base0 + NFULL * CH, TAIL)], idx_t)
    pltpu.sync_copy(ones_v.at[pl.ds(0, TAIL)], cnt_sh.at[idx_t], add=True)
    plsc.subcore_barrier()
    pltpu.sync_copy(cnt_sh.at[pl.ds(row0, RPT)], out_hbm.at[c, pl.ds(row0, RPT)])


@functools.partial(
    pl.kernel,
    out_type=jax.ShapeDtypeStruct((NC, NP, D), jnp.float32),
    mesh=_sc_mesh,
    scratch_types=dict(
        acc_sh=pltpu.VMEM_SHARED((NP, D), jnp.float32),
        rows_v=pltpu.VMEM((CH, D), jnp.float32),
        rows_t=pltpu.VMEM((TAIL, D), jnp.float32),
        sidx_v=pltpu.VMEM((CH,), jnp.int32),
        didx_v=pltpu.VMEM((CH,), jnp.int32),
        sidx_t=pltpu.VMEM((TAIL,), jnp.int32),
        didx_t=pltpu.VMEM((TAIL,), jnp.int32),
        zrow_v=pltpu.VMEM((ZR, D), jnp.float32),
        sem=pltpu.SemaphoreType.DMA,
    ),
)
def _sc_scatter(g_hbm, src_hbm, dst_hbm, out_hbm, acc_sh, rows_v, rows_t,
                sidx_v, didx_v, sidx_t, didx_t, zrow_v, sem):
    c = lax.axis_index("c")
    s = lax.axis_index("s")
    wid = c * NS + s
    _fill_const(zrow_v, ZR, D, 0.0)
    row0 = s * RPT
    for j in range(RPT // ZR):
        pltpu.sync_copy(zrow_v, acc_sh.at[pl.ds(row0 + j * ZR, ZR)])
    plsc.subcore_barrier()
    base0 = wid * EPT

    def chunk(i, _):
        b = base0 + i * CH
        pltpu.sync_copy(src_hbm.at[pl.ds(b, CH)], sidx_v)
        pltpu.sync_copy(dst_hbm.at[pl.ds(b, CH)], didx_v)
        pltpu.async_copy(g_hbm.at[sidx_v], rows_v, sem).wait()
        pltpu.sync_copy(rows_v, acc_sh.at[didx_v], add=True)
        return 0

    lax.fori_loop(0, NFULL, chunk, 0)
    bt = base0 + NFULL * CH
    pltpu.sync_copy(src_hbm.at[pl.ds(bt, TAIL)], sidx_t)
    pltpu.sync_copy(dst_hbm.at[pl.ds(bt, TAIL)], didx_t)
    pltpu.async_copy(g_hbm.at[sidx_t], rows_t, sem).wait()
    pltpu.sync_copy(rows_t, acc_sh.at[didx_t], add=True)
    plsc.subcore_barrier()
    pltpu.sync_copy(acc_sh.at[pl.ds(row0, RPT)], out_hbm.at[c, pl.ds(row0, RPT)])


_BM = 2048
_GRID = NP // _BM


def _dinv_body(degp_ref, out_ref):
    deg = degp_ref[0] + degp_ref[1] + 1.0
    out_ref[...] = lax.rsqrt(deg)


def _tc_dinv(degp):
    return pl.pallas_call(
        _dinv_body,
        out_shape=jax.ShapeDtypeStruct((NP, 16), jnp.float32),
    )(degp)


def _mm1_body(x_ref, w_ref, dinv_ref, out_ref):
    di = dinv_ref[:, 0:1]
    out_ref[...] = jnp.dot(
        x_ref[...], w_ref[...], preferred_element_type=jnp.float32) * di


def _tc_mm1(x, W1, dinv16):
    return pl.pallas_call(
        _mm1_body,
        grid=(_GRID,),
        in_specs=[
            pl.BlockSpec((_BM, D), lambda i: (i, 0)),
            pl.BlockSpec((D, D), lambda i: (0, 0)),
            pl.BlockSpec((_BM, 16), lambda i: (i, 0)),
        ],
        out_specs=pl.BlockSpec((_BM, D), lambda i: (i, 0)),
        out_shape=jax.ShapeDtypeStruct((NP, D), jnp.float32),
    )(x, W1, dinv16)


def _mm2_body(s_ref, g_ref, dinv_ref, b_ref, w_ref, out_ref):
    di = dinv_ref[:, 0:1]
    pre = (s_ref[0] + s_ref[1] + g_ref[...]) * di + b_ref[...]
    h = jnp.maximum(pre, 0.0)
    out_ref[...] = jnp.dot(
        h, w_ref[...], preferred_element_type=jnp.float32) * di


def _tc_mm2(S1, g1, dinv16, b, W):
    return pl.pallas_call(
        _mm2_body,
        grid=(_GRID,),
        in_specs=[
            pl.BlockSpec((NC, _BM, D), lambda i: (0, i, 0)),
            pl.BlockSpec((_BM, D), lambda i: (i, 0)),
            pl.BlockSpec((_BM, 16), lambda i: (i, 0)),
            pl.BlockSpec((1, D), lambda i: (0, 0)),
            pl.BlockSpec((D, D), lambda i: (0, 0)),
        ],
        out_specs=pl.BlockSpec((_BM, D), lambda i: (i, 0)),
        out_shape=jax.ShapeDtypeStruct((NP, D), jnp.float32),
    )(S1, g1, dinv16, b, W)


def _mm3_body(s_ref, g_ref, dinv_ref, b_ref, wfc_ref, bfc_ref, out_ref):
    di = dinv_ref[:, 0:1]
    pre = (s_ref[0] + s_ref[1] + g_ref[...]) * di + b_ref[...]
    h = jnp.maximum(pre, 0.0)
    z = jnp.dot(h, wfc_ref[...], preferred_element_type=jnp.float32)
    out_ref[...] = jax.nn.sigmoid(z + bfc_ref[...])


def _tc_mm3(S2, g2, dinv16, b, Wfc, bfc):
    return pl.pallas_call(
        _mm3_body,
        grid=(_GRID,),
        in_specs=[
            pl.BlockSpec((NC, _BM, D), lambda i: (0, i, 0)),
            pl.BlockSpec((_BM, D), lambda i: (i, 0)),
            pl.BlockSpec((_BM, 16), lambda i: (i, 0)),
            pl.BlockSpec((1, D), lambda i: (0, 0)),
            pl.BlockSpec((D, 1), lambda i: (0, 0)),
            pl.BlockSpec((1, 1), lambda i: (0, 0)),
        ],
        out_specs=pl.BlockSpec((_BM, 1), lambda i: (i, 0)),
        out_shape=jax.ShapeDtypeStruct((NP, 1), jnp.float32),
    )(S2, g2, dinv16, b, Wfc, bfc)


def kernel(x, edge_index, W1, b1, W2, b2, Wfc, bfc):
    ei = edge_index.astype(jnp.int32)
    src = ei[0]
    dst = ei[1]
    xp = jnp.pad(x, ((0, NP - N), (0, 0)))
    degp = _sc_count(dst)
    dinv16 = _tc_dinv(degp)
    g1 = _tc_mm1(xp, W1, dinv16)
    S1 = _sc_scatter(g1, src, dst)
    g2 = _tc_mm2(S1, g1, dinv16, b1.reshape(1, D), W2)
    S2 = _sc_scatter(g2, src, dst)
    out = _tc_mm3(S2, g2, dinv16, b2.reshape(1, D), Wfc, bfc.reshape(1, 1))
    return out[:N]


# speedup vs baseline: 33.2680x; 2.0622x over previous
import functools

import jax
import jax.numpy as jnp
from jax import lax
from jax.experimental import pallas as pl
from jax.experimental.pallas import tpu as pltpu
from jax.experimental.pallas import tpu_sc as plsc

N = 10000
NP = 10240
E = 320000
D = 128

NC = 2
NS = 16
NW = NC * NS

EPT = E // NW
CH = 64
NCHUNK = E // CH

RPT = NP // NS
ZR = 128

_sc_mesh = plsc.VectorSubcoreMesh(
    core_axis_name="c", subcore_axis_name="s", num_cores=NC, num_subcores=NS)


def _fill_const(ref, nrows, ncols, value):
    vals = jnp.full((16,), value, jnp.float32)

    def body(i, _):
        for j in range(ncols // 16):
            ref[i, pl.ds(16 * j, 16)] = vals
        return 0

    lax.fori_loop(0, nrows, body, 0)


KC = 4
CGROUPS = 41


@functools.partial(
    pl.kernel,
    out_type=jax.ShapeDtypeStruct((NC, NP, 16), jnp.float32),
    mesh=_sc_mesh,
    scratch_types=dict(
        cnt_sh=pltpu.VMEM_SHARED((NP, 16), jnp.float32),
        ones_v=pltpu.VMEM((CH, 16), jnp.float32),
        didx0=pltpu.VMEM((CH,), jnp.int32),
        didx1=pltpu.VMEM((CH,), jnp.int32),
        didx2=pltpu.VMEM((CH,), jnp.int32),
        didx3=pltpu.VMEM((CH,), jnp.int32),
        zrow_v=pltpu.VMEM((RPT, 16), jnp.float32),
        isems=pltpu.SemaphoreType.DMA((KC,)),
        ssems=pltpu.SemaphoreType.DMA((KC,)),
    ),
)
def _sc_count(dst_hbm, out_hbm, cnt_sh, ones_v, didx0, didx1, didx2, didx3,
              zrow_v, isems, ssems):
    didx = [didx0, didx1, didx2, didx3]
    c = lax.axis_index("c")
    sub = lax.axis_index("s")
    wid = c * NS + sub
    nch = jnp.where(wid < NCHSPLIT, NCHHI, NCHLO)
    _fill_const(ones_v, CH, 16, 1.0)
    _fill_const(zrow_v, RPT, 16, 0.0)
    row0 = sub * RPT
    pltpu.sync_copy(zrow_v, cnt_sh.at[pl.ds(row0, RPT)])
    plsc.subcore_barrier()

    def group(g, _):
        for k in range(KC):
            st = g * KC + k
            ks = (k + KC - 2) % KC

            @pl.when(jnp.logical_and(st >= KC, st - KC < nch))
            def _():
                pltpu.make_async_copy(
                    ones_v, cnt_sh.at[didx[k]], ssems.at[k]).wait()

            @pl.when(st < nch)
            def _():
                b = (st * NW + wid) * CH
                pltpu.async_copy(dst_hbm.at[pl.ds(b, CH)], didx[k],
                                 isems.at[k])

            @pl.when(jnp.logical_and(st >= 2, st - 2 < nch))
            def _():
                b = ((st - 2) * NW + wid) * CH
                pltpu.make_async_copy(dst_hbm.at[pl.ds(b, CH)],
                                      didx[ks], isems.at[ks]).wait()
                pltpu.async_copy(ones_v, cnt_sh.at[didx[ks]],
                                 ssems.at[ks], add=True)
        return 0

    lax.fori_loop(0, CGROUPS, group, 0)
    plsc.subcore_barrier()
    pltpu.sync_copy(cnt_sh.at[pl.ds(row0, RPT)], out_hbm.at[c, pl.ds(row0, RPT)])


K = 5
NCHHI = 157
NCHLO = 156
NCHSPLIT = 8
GROUPS = 33


def _zero_rows(ref, k, nrows, ncols):
    z = jnp.zeros((16,), jnp.float32)

    def body(i, _):
        for j in range(ncols // 16):
            ref[k, i, pl.ds(16 * j, 16)] = z
        return 0

    lax.fori_loop(0, nrows, body, 0)


@functools.partial(
    pl.kernel,
    out_type=jax.ShapeDtypeStruct((NC, NP, D), jnp.float32),
    mesh=_sc_mesh,
    scratch_types=dict(
        acc_sh=pltpu.VMEM_SHARED((NP, D), jnp.float32),
        rows_v=pltpu.VMEM((K, CH, D), jnp.float32),
        sidx0=pltpu.VMEM((CH,), jnp.int32),
        sidx1=pltpu.VMEM((CH,), jnp.int32),
        sidx2=pltpu.VMEM((CH,), jnp.int32),
        sidx3=pltpu.VMEM((CH,), jnp.int32),
        sidx4=pltpu.VMEM((CH,), jnp.int32),
        didx0=pltpu.VMEM((CH,), jnp.int32),
        didx1=pltpu.VMEM((CH,), jnp.int32),
        didx2=pltpu.VMEM((CH,), jnp.int32),
        didx3=pltpu.VMEM((CH,), jnp.int32),
        didx4=pltpu.VMEM((CH,), jnp.int32),
        isems=pltpu.SemaphoreType.DMA((K,)),
        idems=pltpu.SemaphoreType.DMA((K,)),
        gsems=pltpu.SemaphoreType.DMA((K,)),
        ssems=pltpu.SemaphoreType.DMA((K,)),
    ),
)
def _sc_scatter(g_hbm, src_hbm, dst_hbm, out_hbm, acc_sh, rows_v,
                sidx0, sidx1, sidx2, sidx3, sidx4,
                didx0, didx1, didx2, didx3, didx4,
                isems, idems, gsems, ssems):
    sidx = [sidx0, sidx1, sidx2, sidx3, sidx4]
    didx = [didx0, didx1, didx2, didx3, didx4]
    c = lax.axis_index("c")
    sub = lax.axis_index("s")
    wid = c * NS + sub
    nch = jnp.where(wid < NCHSPLIT, NCHHI, NCHLO)
    _zero_rows(rows_v, 0, CH, D)
    row0 = sub * RPT
    for j in range(RPT // CH):
        pltpu.sync_copy(rows_v.at[0], acc_sh.at[pl.ds(row0 + j * CH, CH)])
    plsc.subcore_barrier()

    def group(g, _):
        for k in range(K):
            st = g * K + k
            kg = (k + K - 2) % K
            ks = (k + K - 4) % K

            @pl.when(jnp.logical_and(st >= K, st - K < nch))
            def _():
                pltpu.make_async_copy(
                    rows_v.at[k], acc_sh.at[didx[k]], ssems.at[k]).wait()

            @pl.when(st < nch)
            def _():
                b = (st * NW + wid) * CH
                pltpu.async_copy(src_hbm.at[pl.ds(b, CH)], sidx[k],
                                 isems.at[k])
                pltpu.async_copy(dst_hbm.at[pl.ds(b, CH)], didx[k],
                                 idems.at[k])

            @pl.when(jnp.logical_and(st >= 2, st - 2 < nch))
            def _():
                b = ((st - 2) * NW + wid) * CH
                pltpu.make_async_copy(src_hbm.at[pl.ds(b, CH)],
                                      sidx[kg], isems.at[kg]).wait()
                pltpu.make_async_copy(dst_hbm.at[pl.ds(b, CH)],
                                      didx[kg], idems.at[kg]).wait()
                pltpu.async_copy(g_hbm.at[sidx[kg]], rows_v.at[kg],
                                 gsems.at[kg])

            @pl.when(jnp.logical_and(st >= 4, st - 4 < nch))
            def _():
                pltpu.make_async_copy(g_hbm.at[sidx[ks]],
                                      rows_v.at[ks], gsems.at[ks]).wait()
                pltpu.async_copy(rows_v.at[ks], acc_sh.at[didx[ks]],
                                 ssems.at[ks], add=True)
        return 0

    lax.fori_loop(0, GROUPS, group, 0)
    plsc.subcore_barrier()
    pltpu.sync_copy(acc_sh.at[pl.ds(row0, RPT)], out_hbm.at[c, pl.ds(row0, RPT)])


_BM = 2048
_GRID = NP // _BM


def _mmA_body(x_ref, w_ref, out_ref):
    out_ref[...] = jnp.dot(
        x_ref[...], w_ref[...], preferred_element_type=jnp.float32)


def _tc_mmA(x, W1):
    return pl.pallas_call(
        _mmA_body,
        grid=(_GRID,),
        in_specs=[
            pl.BlockSpec((_BM, D), lambda i: (i, 0)),
            pl.BlockSpec((D, D), lambda i: (0, 0)),
        ],
        out_specs=pl.BlockSpec((_BM, D), lambda i: (i, 0)),
        out_shape=jax.ShapeDtypeStruct((NP, D), jnp.float32),
    )(x, W1)


def _scale1_body(h_ref, degp_ref, g_ref, dinv_ref):
    dinv = lax.rsqrt(degp_ref[0] + degp_ref[1] + 1.0)
    dinv_ref[...] = dinv
    g_ref[...] = h_ref[...] * dinv[:, 0:1]


def _tc_scale1(h1x, degp):
    return pl.pallas_call(
        _scale1_body,
        grid=(_GRID,),
        in_specs=[
            pl.BlockSpec((_BM, D), lambda i: (i, 0)),
            pl.BlockSpec((NC, _BM, 16), lambda i: (0, i, 0)),
        ],
        out_specs=[
            pl.BlockSpec((_BM, D), lambda i: (i, 0)),
            pl.BlockSpec((_BM, 16), lambda i: (i, 0)),
        ],
        out_shape=[
            jax.ShapeDtypeStruct((NP, D), jnp.float32),
            jax.ShapeDtypeStruct((NP, 16), jnp.float32),
        ],
    )(h1x, degp)


def _mm2_body(s_ref, g_ref, dinv_ref, b_ref, w_ref, out_ref):
    di = dinv_ref[:, 0:1]
    pre = (s_ref[0] + s_ref[1] + g_ref[...]) * di + b_ref[...]
    h = jnp.maximum(pre, 0.0)
    out_ref[...] = jnp.dot(
        h, w_ref[...], preferred_element_type=jnp.float32) * di


def _tc_mm2(S1, g1, dinv16, b, W):
    return pl.pallas_call(
        _mm2_body,
        grid=(_GRID,),
        in_specs=[
            pl.BlockSpec((NC, _BM, D), lambda i: (0, i, 0)),
            pl.BlockSpec((_BM, D), lambda i: (i, 0)),
            pl.BlockSpec((_BM, 16), lambda i: (i, 0)),
            pl.BlockSpec((1, D), lambda i: (0, 0)),
            pl.BlockSpec((D, D), lambda i: (0, 0)),
        ],
        out_specs=pl.BlockSpec((_BM, D), lambda i: (i, 0)),
        out_shape=jax.ShapeDtypeStruct((NP, D), jnp.float32),
    )(S1, g1, dinv16, b, W)


def _mm3_body(s_ref, g_ref, dinv_ref, b_ref, wfc_ref, bfc_ref, out_ref):
    di = dinv_ref[:, 0:1]
    pre = (s_ref[0] + s_ref[1] + g_ref[...]) * di + b_ref[...]
    h = jnp.maximum(pre, 0.0)
    z = jnp.dot(h, wfc_ref[...], preferred_element_type=jnp.float32)
    out_ref[...] = jax.nn.sigmoid(z + bfc_ref[...])


def _tc_mm3(S2, g2, dinv16, b, Wfc, bfc):
    return pl.pallas_call(
        _mm3_body,
        grid=(_GRID,),
        in_specs=[
            pl.BlockSpec((NC, _BM, D), lambda i: (0, i, 0)),
            pl.BlockSpec((_BM, D), lambda i: (i, 0)),
            pl.BlockSpec((_BM, 16), lambda i: (i, 0)),
            pl.BlockSpec((1, D), lambda i: (0, 0)),
            pl.BlockSpec((D, 1), lambda i: (0, 0)),
            pl.BlockSpec((1, 1), lambda i: (0, 0)),
        ],
        out_specs=pl.BlockSpec((_BM, 1), lambda i: (i, 0)),
        out_shape=jax.ShapeDtypeStruct((NP, 1), jnp.float32),
    )(S2, g2, dinv16, b, Wfc, bfc)


def kernel(x, edge_index, W1, b1, W2, b2, Wfc, bfc):
    ei = edge_index.astype(jnp.int32)
    src = ei[0]
    dst = ei[1]
    xp = jnp.pad(x, ((0, NP - N), (0, 0)))
    degp = _sc_count(dst)
    h1x = _tc_mmA(xp, W1)
    g1, dinv16 = _tc_scale1(h1x, degp)
    S1 = _sc_scatter(g1, src, dst)
    g2 = _tc_mm2(S1, g1, dinv16, b1.reshape(1, D), W2)
    S2 = _sc_scatter(g2, src, dst)
    out = _tc_mm3(S2, g2, dinv16, b2.reshape(1, D), Wfc, bfc.reshape(1, 1))
    return out[:N]


# speedup vs baseline: 34.6158x; 1.0405x over previous
import functools

import jax
import jax.numpy as jnp
from jax import lax
from jax.experimental import pallas as pl
from jax.experimental.pallas import tpu as pltpu
from jax.experimental.pallas import tpu_sc as plsc

N = 10000
NP = 10240
E = 320000
D = 128

NC = 2
NS = 16
NW = NC * NS

EPT = E // NW
CH = 64
NCHUNK = E // CH

RPT = NP // NS
ZR = 128

_sc_mesh = plsc.VectorSubcoreMesh(
    core_axis_name="c", subcore_axis_name="s", num_cores=NC, num_subcores=NS)


def _fill_const(ref, nrows, ncols, value):
    vals = jnp.full((16,), value, jnp.float32)

    def body(i, _):
        for j in range(ncols // 16):
            ref[i, pl.ds(16 * j, 16)] = vals
        return 0

    lax.fori_loop(0, nrows, body, 0)


KC = 6
CCH = 128
CNHI = 79
CNLO = 78
CSPLIT = 4
CGROUPS = 15


@functools.partial(
    pl.kernel,
    out_type=jax.ShapeDtypeStruct((NC, NP, 16), jnp.float32),
    mesh=_sc_mesh,
    scratch_types=dict(
        cnt_sh=pltpu.VMEM_SHARED((NP, 16), jnp.float32),
        ones_v=pltpu.VMEM((CCH, 16), jnp.float32),
        didx0=pltpu.VMEM((CCH,), jnp.int32),
        didx1=pltpu.VMEM((CCH,), jnp.int32),
        didx2=pltpu.VMEM((CCH,), jnp.int32),
        didx3=pltpu.VMEM((CCH,), jnp.int32),
        didx4=pltpu.VMEM((CCH,), jnp.int32),
        didx5=pltpu.VMEM((CCH,), jnp.int32),
        zrow_v=pltpu.VMEM((RPT, 16), jnp.float32),
        isems=pltpu.SemaphoreType.DMA((KC,)),
        ssems=pltpu.SemaphoreType.DMA((KC,)),
    ),
)
def _sc_count(dst_hbm, out_hbm, cnt_sh, ones_v, didx0, didx1, didx2, didx3,
              didx4, didx5, zrow_v, isems, ssems):
    didx = [didx0, didx1, didx2, didx3, didx4, didx5]
    c = lax.axis_index("c")
    sub = lax.axis_index("s")
    wid = c * NS + sub
    nch = jnp.where(wid < CSPLIT, CNHI, CNLO)
    _fill_const(ones_v, CCH, 16, 1.0)
    _fill_const(zrow_v, RPT, 16, 0.0)
    row0 = sub * RPT
    pltpu.sync_copy(zrow_v, cnt_sh.at[pl.ds(row0, RPT)])
    plsc.subcore_barrier()

    def group(g, _):
        for k in range(KC):
            st = g * KC + k
            ks = (k + KC - 2) % KC

            @pl.when(jnp.logical_and(st >= KC, st - KC < nch))
            def _():
                pltpu.make_async_copy(
                    ones_v, cnt_sh.at[didx[k]], ssems.at[k]).wait()

            @pl.when(st < nch)
            def _():
                b = (st * NW + wid) * CCH
                pltpu.async_copy(dst_hbm.at[pl.ds(b, CCH)], didx[k],
                                 isems.at[k])

            @pl.when(jnp.logical_and(st >= 2, st - 2 < nch))
            def _():
                b = ((st - 2) * NW + wid) * CCH
                pltpu.make_async_copy(dst_hbm.at[pl.ds(b, CCH)],
                                      didx[ks], isems.at[ks]).wait()
                pltpu.async_copy(ones_v, cnt_sh.at[didx[ks]],
                                 ssems.at[ks], add=True)
        return 0

    lax.fori_loop(0, CGROUPS, group, 0)
    plsc.subcore_barrier()
    pltpu.sync_copy(cnt_sh.at[pl.ds(row0, RPT)], out_hbm.at[c, pl.ds(row0, RPT)])


K = 5
NCHHI = 157
NCHLO = 156
NCHSPLIT = 8
GROUPS = 33


def _zero_rows(ref, k, nrows, ncols):
    z = jnp.zeros((16,), jnp.float32)

    def body(i, _):
        for j in range(ncols // 16):
            ref[k, i, pl.ds(16 * j, 16)] = z
        return 0

    lax.fori_loop(0, nrows, body, 0)


@functools.partial(
    pl.kernel,
    out_type=jax.ShapeDtypeStruct((NC, NP, D), jnp.float32),
    mesh=_sc_mesh,
    scratch_types=dict(
        acc_sh=pltpu.VMEM_SHARED((NP, D), jnp.float32),
        rows_v=pltpu.VMEM((K, CH, D), jnp.float32),
        sidx0=pltpu.VMEM((CH,), jnp.int32),
        sidx1=pltpu.VMEM((CH,), jnp.int32),
        sidx2=pltpu.VMEM((CH,), jnp.int32),
        sidx3=pltpu.VMEM((CH,), jnp.int32),
        sidx4=pltpu.VMEM((CH,), jnp.int32),
        didx0=pltpu.VMEM((CH,), jnp.int32),
        didx1=pltpu.VMEM((CH,), jnp.int32),
        didx2=pltpu.VMEM((CH,), jnp.int32),
        didx3=pltpu.VMEM((CH,), jnp.int32),
        didx4=pltpu.VMEM((CH,), jnp.int32),
        isems=pltpu.SemaphoreType.DMA((K,)),
        idems=pltpu.SemaphoreType.DMA((K,)),
        gsems=pltpu.SemaphoreType.DMA((K,)),
        ssems=pltpu.SemaphoreType.DMA((K,)),
    ),
)
def _sc_scatter(g_hbm, src_hbm, dst_hbm, out_hbm, acc_sh, rows_v,
                sidx0, sidx1, sidx2, sidx3, sidx4,
                didx0, didx1, didx2, didx3, didx4,
                isems, idems, gsems, ssems):
    sidx = [sidx0, sidx1, sidx2, sidx3, sidx4]
    didx = [didx0, didx1, didx2, didx3, didx4]
    c = lax.axis_index("c")
    sub = lax.axis_index("s")
    wid = c * NS + sub
    nch = jnp.where(wid < NCHSPLIT, NCHHI, NCHLO)
    _zero_rows(rows_v, 0, CH, D)
    row0 = sub * RPT
    for j in range(RPT // CH):
        pltpu.async_copy(rows_v.at[0], acc_sh.at[pl.ds(row0 + j * CH, CH)],
                         ssems.at[0])
    for j in range(RPT // CH):
        pltpu.make_async_copy(rows_v.at[0],
                              acc_sh.at[pl.ds(row0 + j * CH, CH)],
                              ssems.at[0]).wait()
    plsc.subcore_barrier()

    def group(g, _):
        for k in range(K):
            st = g * K + k
            kg = (k + K - 2) % K
            ks = (k + K - 4) % K

            @pl.when(jnp.logical_and(st >= K, st - K < nch))
            def _():
                pltpu.make_async_copy(
                    rows_v.at[k], acc_sh.at[didx[k]], ssems.at[k]).wait()

            @pl.when(st < nch)
            def _():
                b = (st * NW + wid) * CH
                pltpu.async_copy(src_hbm.at[pl.ds(b, CH)], sidx[k],
                                 isems.at[k])
                pltpu.async_copy(dst_hbm.at[pl.ds(b, CH)], didx[k],
                                 idems.at[k])

            @pl.when(jnp.logical_and(st >= 2, st - 2 < nch))
            def _():
                b = ((st - 2) * NW + wid) * CH
                pltpu.make_async_copy(src_hbm.at[pl.ds(b, CH)],
                                      sidx[kg], isems.at[kg]).wait()
                pltpu.make_async_copy(dst_hbm.at[pl.ds(b, CH)],
                                      didx[kg], idems.at[kg]).wait()
                pltpu.async_copy(g_hbm.at[sidx[kg]], rows_v.at[kg],
                                 gsems.at[kg])

            @pl.when(jnp.logical_and(st >= 4, st - 4 < nch))
            def _():
                pltpu.make_async_copy(g_hbm.at[sidx[ks]],
                                      rows_v.at[ks], gsems.at[ks]).wait()
                pltpu.async_copy(rows_v.at[ks], acc_sh.at[didx[ks]],
                                 ssems.at[ks], add=True)
        return 0

    lax.fori_loop(0, GROUPS, group, 0)
    plsc.subcore_barrier()
    pltpu.sync_copy(acc_sh.at[pl.ds(row0, RPT)], out_hbm.at[c, pl.ds(row0, RPT)])


_BM = 2048
_GRID = NP // _BM


def _scale0_body(x_ref, degp_ref, g_ref, dinv_ref):
    dinv = lax.rsqrt(degp_ref[0] + degp_ref[1] + 1.0)
    dinv_ref[...] = dinv
    g_ref[...] = x_ref[...] * dinv[:, 0:1]


def _tc_scale0(x, degp):
    return pl.pallas_call(
        _scale0_body,
        grid=(_GRID,),
        in_specs=[
            pl.BlockSpec((_BM, D), lambda i: (i, 0)),
            pl.BlockSpec((NC, _BM, 16), lambda i: (0, i, 0)),
        ],
        out_specs=[
            pl.BlockSpec((_BM, D), lambda i: (i, 0)),
            pl.BlockSpec((_BM, 16), lambda i: (i, 0)),
        ],
        out_shape=[
            jax.ShapeDtypeStruct((NP, D), jnp.float32),
            jax.ShapeDtypeStruct((NP, 16), jnp.float32),
        ],
    )(x, degp)


def _mid_body(s_ref, g_ref, dinv_ref, b_ref, w_ref, out_ref):
    di = dinv_ref[:, 0:1]
    agg = s_ref[0] + s_ref[1] + g_ref[...]
    pre = jnp.dot(agg, w_ref[...], preferred_element_type=jnp.float32)
    h = jnp.maximum(pre * di + b_ref[...], 0.0)
    out_ref[...] = h * di


def _tc_mid(S, gt, dinv16, b, W):
    return pl.pallas_call(
        _mid_body,
        grid=(_GRID,),
        in_specs=[
            pl.BlockSpec((NC, _BM, D), lambda i: (0, i, 0)),
            pl.BlockSpec((_BM, D), lambda i: (i, 0)),
            pl.BlockSpec((_BM, 16), lambda i: (i, 0)),
            pl.BlockSpec((1, D), lambda i: (0, 0)),
            pl.BlockSpec((D, D), lambda i: (0, 0)),
        ],
        out_specs=pl.BlockSpec((_BM, D), lambda i: (i, 0)),
        out_shape=jax.ShapeDtypeStruct((NP, D), jnp.float32),
    )(S, gt, dinv16, b, W)


def _fin_body(s_ref, g_ref, dinv_ref, b_ref, w_ref, wfc_ref, bfc_ref, out_ref):
    di = dinv_ref[:, 0:1]
    agg = s_ref[0] + s_ref[1] + g_ref[...]
    pre = jnp.dot(agg, w_ref[...], preferred_element_type=jnp.float32)
    h = jnp.maximum(pre * di + b_ref[...], 0.0)
    z = jnp.dot(h, wfc_ref[...], preferred_element_type=jnp.float32)
    out_ref[...] = jax.nn.sigmoid(z + bfc_ref[...])


def _tc_fin(S, gt, dinv16, b, W, Wfc, bfc):
    return pl.pallas_call(
        _fin_body,
        grid=(_GRID,),
        in_specs=[
            pl.BlockSpec((NC, _BM, D), lambda i: (0, i, 0)),
            pl.BlockSpec((_BM, D), lambda i: (i, 0)),
            pl.BlockSpec((_BM, 16), lambda i: (i, 0)),
            pl.BlockSpec((1, D), lambda i: (0, 0)),
            pl.BlockSpec((D, D), lambda i: (0, 0)),
            pl.BlockSpec((D, 1), lambda i: (0, 0)),
            pl.BlockSpec((1, 1), lambda i: (0, 0)),
        ],
        out_specs=pl.BlockSpec((_BM, 1), lambda i: (i, 0)),
        out_shape=jax.ShapeDtypeStruct((NP, 1), jnp.float32),
    )(S, gt, dinv16, b, W, Wfc, bfc)


def kernel(x, edge_index, W1, b1, W2, b2, Wfc, bfc):
    ei = edge_index.astype(jnp.int32)
    src = ei[0]
    dst = ei[1]
    xp = jnp.pad(x, ((0, NP - N), (0, 0)))
    degp = _sc_count(dst)
    gt1, dinv16 = _tc_scale0(xp, degp)
    S1 = _sc_scatter(gt1, src, dst)
    gt2 = _tc_mid(S1, gt1, dinv16, b1.reshape(1, D), W1)
    S2 = _sc_scatter(gt2, src, dst)
    out = _tc_fin(S2, gt2, dinv16, b2.reshape(1, D), W2, Wfc, bfc.reshape(1, 1))
    return out[:N]


# speedup vs baseline: 35.9030x; 1.0372x over previous
import functools

import jax
import jax.numpy as jnp
from jax import lax
from jax.experimental import pallas as pl
from jax.experimental.pallas import tpu as pltpu
from jax.experimental.pallas import tpu_sc as plsc

N = 10000
NP = 10240
E = 320000
D = 128

NC = 2
NS = 16
NW = NC * NS

EPT = E // NW
CH = 64
NCHUNK = E // CH

RPT = NP // NS
ZR = 128

_sc_mesh = plsc.VectorSubcoreMesh(
    core_axis_name="c", subcore_axis_name="s", num_cores=NC, num_subcores=NS)


def _fill_const(ref, nrows, ncols, value):
    vals = jnp.full((16,), value, jnp.float32)

    def body(i, _):
        for j in range(ncols // 16):
            ref[i, pl.ds(16 * j, 16)] = vals
        return 0

    lax.fori_loop(0, nrows, body, 0)


KC = 6
CCH = 128
CNHI = 79
CNLO = 78
CSPLIT = 4
CGROUPS = 15


@functools.partial(
    pl.kernel,
    out_type=jax.ShapeDtypeStruct((NC, NP, 16), jnp.float32),
    mesh=_sc_mesh,
    scratch_types=dict(
        cnt_sh=pltpu.VMEM_SHARED((NP, 16), jnp.float32),
        ones_v=pltpu.VMEM((CCH, 16), jnp.float32),
        didx0=pltpu.VMEM((CCH,), jnp.int32),
        didx1=pltpu.VMEM((CCH,), jnp.int32),
        didx2=pltpu.VMEM((CCH,), jnp.int32),
        didx3=pltpu.VMEM((CCH,), jnp.int32),
        didx4=pltpu.VMEM((CCH,), jnp.int32),
        didx5=pltpu.VMEM((CCH,), jnp.int32),
        zrow_v=pltpu.VMEM((RPT, 16), jnp.float32),
        isems=pltpu.SemaphoreType.DMA((KC,)),
        ssems=pltpu.SemaphoreType.DMA((KC,)),
    ),
)
def _sc_count(dst_hbm, out_hbm, cnt_sh, ones_v, didx0, didx1, didx2, didx3,
              didx4, didx5, zrow_v, isems, ssems):
    didx = [didx0, didx1, didx2, didx3, didx4, didx5]
    c = lax.axis_index("c")
    sub = lax.axis_index("s")
    wid = c * NS + sub
    nch = jnp.where(wid < CSPLIT, CNHI, CNLO)
    _fill_const(ones_v, CCH, 16, 1.0)
    _fill_const(zrow_v, RPT, 16, 0.0)
    row0 = sub * RPT
    pltpu.sync_copy(zrow_v, cnt_sh.at[pl.ds(row0, RPT)])
    plsc.subcore_barrier()

    def group(g, _):
        for k in range(KC):
            st = g * KC + k
            ks = (k + KC - 2) % KC

            @pl.when(jnp.logical_and(st >= KC, st - KC < nch))
            def _():
                pltpu.make_async_copy(
                    ones_v, cnt_sh.at[didx[k]], ssems.at[k]).wait()

            @pl.when(st < nch)
            def _():
                b = (st * NW + wid) * CCH
                pltpu.async_copy(dst_hbm.at[pl.ds(b, CCH)], didx[k],
                                 isems.at[k])

            @pl.when(jnp.logical_and(st >= 2, st - 2 < nch))
            def _():
                b = ((st - 2) * NW + wid) * CCH
                pltpu.make_async_copy(dst_hbm.at[pl.ds(b, CCH)],
                                      didx[ks], isems.at[ks]).wait()
                pltpu.async_copy(ones_v, cnt_sh.at[didx[ks]],
                                 ssems.at[ks], add=True)
        return 0

    lax.fori_loop(0, CGROUPS, group, 0)
    plsc.subcore_barrier()
    pltpu.sync_copy(cnt_sh.at[pl.ds(row0, RPT)], out_hbm.at[c, pl.ds(row0, RPT)])


K = 5
NCHHI = 157
NCHLO = 156
NCHSPLIT = 8
GROUPS = 33


def _zero_rows(ref, k, nrows, ncols):
    z = jnp.zeros((16,), jnp.float32)

    def body(i, _):
        for j in range(ncols // 16):
            ref[k, i, pl.ds(16 * j, 16)] = z
        return 0

    lax.fori_loop(0, nrows, body, 0)


@functools.partial(
    pl.kernel,
    out_type=jax.ShapeDtypeStruct((NC, NP, D), jnp.float32),
    mesh=_sc_mesh,
    scratch_types=dict(
        acc_sh=pltpu.VMEM_SHARED((NP, D), jnp.float32),
        rows_v=pltpu.VMEM((K, CH, D), jnp.float32),
        sidx0=pltpu.VMEM((CH,), jnp.int32),
        sidx1=pltpu.VMEM((CH,), jnp.int32),
        sidx2=pltpu.VMEM((CH,), jnp.int32),
        sidx3=pltpu.VMEM((CH,), jnp.int32),
        sidx4=pltpu.VMEM((CH,), jnp.int32),
        didx0=pltpu.VMEM((CH,), jnp.int32),
        didx1=pltpu.VMEM((CH,), jnp.int32),
        didx2=pltpu.VMEM((CH,), jnp.int32),
        didx3=pltpu.VMEM((CH,), jnp.int32),
        didx4=pltpu.VMEM((CH,), jnp.int32),
        isems=pltpu.SemaphoreType.DMA((K,)),
        idems=pltpu.SemaphoreType.DMA((K,)),
        gsems=pltpu.SemaphoreType.DMA((K,)),
        ssems=pltpu.SemaphoreType.DMA((K,)),
    ),
)
def _sc_scatter(g_hbm, src_hbm, dst_hbm, out_hbm, acc_sh, rows_v,
                sidx0, sidx1, sidx2, sidx3, sidx4,
                didx0, didx1, didx2, didx3, didx4,
                isems, idems, gsems, ssems):
    sidx = [sidx0, sidx1, sidx2, sidx3, sidx4]
    didx = [didx0, didx1, didx2, didx3, didx4]
    c = lax.axis_index("c")
    sub = lax.axis_index("s")
    wid = c * NS + sub
    nch = jnp.where(wid < NCHSPLIT, NCHHI, NCHLO)
    _zero_rows(rows_v, 0, CH, D)
    row0 = sub * RPT
    for j in range(RPT // CH):
        pltpu.async_copy(rows_v.at[0], acc_sh.at[pl.ds(row0 + j * CH, CH)],
                         ssems.at[0])
    for j in range(RPT // CH):
        pltpu.make_async_copy(rows_v.at[0],
                              acc_sh.at[pl.ds(row0 + j * CH, CH)],
                              ssems.at[0]).wait()
    plsc.subcore_barrier()

    def group(g, _):
        for k in range(K):
            st = g * K + k
            kg = (k + K - 1) % K
            ks = (k + K - 4) % K

            @pl.when(jnp.logical_and(st >= K, st - K < nch))
            def _():
                pltpu.make_async_copy(
                    rows_v.at[k], acc_sh.at[didx[k]], ssems.at[k]).wait()

            @pl.when(st < nch)
            def _():
                b = (st * NW + wid) * CH
                pltpu.async_copy(src_hbm.at[pl.ds(b, CH)], sidx[k],
                                 isems.at[k])
                pltpu.async_copy(dst_hbm.at[pl.ds(b, CH)], didx[k],
                                 idems.at[k])

            @pl.when(jnp.logical_and(st >= 1, st - 1 < nch))
            def _():
                b = ((st - 1) * NW + wid) * CH
                pltpu.make_async_copy(src_hbm.at[pl.ds(b, CH)],
                                      sidx[kg], isems.at[kg]).wait()
                pltpu.make_async_copy(dst_hbm.at[pl.ds(b, CH)],
                                      didx[kg], idems.at[kg]).wait()
                pltpu.async_copy(g_hbm.at[sidx[kg]], rows_v.at[kg],
                                 gsems.at[kg])

            @pl.when(jnp.logical_and(st >= 4, st - 4 < nch))
            def _():
                pltpu.make_async_copy(g_hbm.at[sidx[ks]],
                                      rows_v.at[ks], gsems.at[ks]).wait()
                pltpu.async_copy(rows_v.at[ks], acc_sh.at[didx[ks]],
                                 ssems.at[ks], add=True)
        return 0

    lax.fori_loop(0, GROUPS, group, 0)
    plsc.subcore_barrier()
    pltpu.sync_copy(acc_sh.at[pl.ds(row0, RPT)], out_hbm.at[c, pl.ds(row0, RPT)])


_BM = 2048
_GRID = NP // _BM


def _scale0_body(x_ref, degp_ref, g_ref, dinv_ref):
    dinv = lax.rsqrt(degp_ref[0] + degp_ref[1] + 1.0)
    dinv_ref[...] = dinv
    g_ref[...] = x_ref[...] * dinv[:, 0:1]


def _tc_scale0(x, degp):
    return pl.pallas_call(
        _scale0_body,
        grid=(_GRID,),
        in_specs=[
            pl.BlockSpec((_BM, D), lambda i: (i, 0)),
            pl.BlockSpec((NC, _BM, 16), lambda i: (0, i, 0)),
        ],
        out_specs=[
            pl.BlockSpec((_BM, D), lambda i: (i, 0)),
            pl.BlockSpec((_BM, 16), lambda i: (i, 0)),
        ],
        out_shape=[
            jax.ShapeDtypeStruct((NP, D), jnp.float32),
            jax.ShapeDtypeStruct((NP, 16), jnp.float32),
        ],
    )(x, degp)


def _mid_body(s_ref, g_ref, dinv_ref, b_ref, w_ref, out_ref):
    di = dinv_ref[:, 0:1]
    agg = s_ref[0] + s_ref[1] + g_ref[...]
    pre = jnp.dot(agg, w_ref[...], preferred_element_type=jnp.float32)
    h = jnp.maximum(pre * di + b_ref[...], 0.0)
    out_ref[...] = h * di


def _tc_mid(S, gt, dinv16, b, W):
    return pl.pallas_call(
        _mid_body,
        grid=(_GRID,),
        in_specs=[
            pl.BlockSpec((NC, _BM, D), lambda i: (0, i, 0)),
            pl.BlockSpec((_BM, D), lambda i: (i, 0)),
            pl.BlockSpec((_BM, 16), lambda i: (i, 0)),
            pl.BlockSpec((1, D), lambda i: (0, 0)),
            pl.BlockSpec((D, D), lambda i: (0, 0)),
        ],
        out_specs=pl.BlockSpec((_BM, D), lambda i: (i, 0)),
        out_shape=jax.ShapeDtypeStruct((NP, D), jnp.float32),
    )(S, gt, dinv16, b, W)


def _fin_body(s_ref, g_ref, dinv_ref, b_ref, w_ref, wfc_ref, bfc_ref, out_ref):
    di = dinv_ref[:, 0:1]
    agg = s_ref[0] + s_ref[1] + g_ref[...]
    pre = jnp.dot(agg, w_ref[...], preferred_element_type=jnp.float32)
    h = jnp.maximum(pre * di + b_ref[...], 0.0)
    z = jnp.dot(h, wfc_ref[...], preferred_element_type=jnp.float32)
    out_ref[...] = jax.nn.sigmoid(z + bfc_ref[...])


def _tc_fin(S, gt, dinv16, b, W, Wfc, bfc):
    return pl.pallas_call(
        _fin_body,
        grid=(_GRID,),
        in_specs=[
            pl.BlockSpec((NC, _BM, D), lambda i: (0, i, 0)),
            pl.BlockSpec((_BM, D), lambda i: (i, 0)),
            pl.BlockSpec((_BM, 16), lambda i: (i, 0)),
            pl.BlockSpec((1, D), lambda i: (0, 0)),
            pl.BlockSpec((D, D), lambda i: (0, 0)),
            pl.BlockSpec((D, 1), lambda i: (0, 0)),
            pl.BlockSpec((1, 1), lambda i: (0, 0)),
        ],
        out_specs=pl.BlockSpec((_BM, 1), lambda i: (i, 0)),
        out_shape=jax.ShapeDtypeStruct((NP, 1), jnp.float32),
    )(S, gt, dinv16, b, W, Wfc, bfc)


def kernel(x, edge_index, W1, b1, W2, b2, Wfc, bfc):
    ei = edge_index.astype(jnp.int32)
    src = ei[0]
    dst = ei[1]
    xp = jnp.pad(x, ((0, NP - N), (0, 0)))
    degp = _sc_count(dst)
    gt1, dinv16 = _tc_scale0(xp, degp)
    S1 = _sc_scatter(gt1, src, dst)
    gt2 = _tc_mid(S1, gt1, dinv16, b1.reshape(1, D), W1)
    S2 = _sc_scatter(gt2, src, dst)
    out = _tc_fin(S2, gt2, dinv16, b2.reshape(1, D), W2, Wfc, bfc.reshape(1, 1))
    return out[:N]
